# Initial kernel scaffold; baseline (speedup 1.0000x reference)
import jax
import jax.numpy as jnp
from jax.experimental import pallas as pl


def kernel(price_data_x, edge_index, news_features, conv_w, conv_b, npw1, npb1, npw2, npb2, gw1, gb1, gw2, gb2, mw1, mb1, mw2, mb2):
    raise NotImplementedError("write your pallas kernel here")


# speedup vs baseline: 5.2911x; 5.2911x over previous
import functools

import jax
import jax.numpy as jnp
from jax import lax
from jax.experimental import pallas as pl
from jax.experimental.pallas import tpu as pltpu
from jax.experimental.pallas import tpu_sc as plsc

B = 2
N = 10000
E = 160000
L = 64
NEWS_DIM = 128
CNN_C = 64
NEWS_P = 32
D1 = 128
D2 = 128
G_HID = 256
NSUB = 16
EPS = E // NSUB
CH = 80
NCHUNK = EPS // CH
RPS = 624
TAIL = N - NSUB * RPS
TOFF = NSUB * RPS


def _seg_sum(table, src_b, dst_loc, d):
    zeros = jnp.zeros((N, d), jnp.float32)
    mesh = plsc.VectorSubcoreMesh(core_axis_name="c", subcore_axis_name="s")

    @functools.partial(
        pl.kernel,
        mesh=mesh,
        out_type=jax.ShapeDtypeStruct((B * N, d), jnp.float32),
        scratch_types=[
            pltpu.VMEM((CH,), jnp.int32),
            pltpu.VMEM((CH,), jnp.int32),
            pltpu.VMEM((CH, d), jnp.float32),
            pltpu.VMEM_SHARED((N, d), jnp.float32),
            pltpu.SemaphoreType.DMA,
        ],
    )
    def k(table_h, srcb_h, dstl_h, zeros_h, out_h, src_v, dst_v, rows_v, acc, sem):
        c = lax.axis_index("c")
        s = lax.axis_index("s")
        pltpu.sync_copy(zeros_h.at[pl.ds(s * RPS, RPS)], acc.at[pl.ds(s * RPS, RPS)])

        @pl.when(s == NSUB - 1)
        def _():
            pltpu.sync_copy(zeros_h.at[pl.ds(TOFF, TAIL)], acc.at[pl.ds(TOFF, TAIL)])

        plsc.subcore_barrier()

        def body(i, carry):
            e0 = s * EPS + i * CH
            pltpu.sync_copy(srcb_h.at[pl.ds(c * E + e0, CH)], src_v)
            pltpu.sync_copy(dstl_h.at[pl.ds(e0, CH)], dst_v)
            pltpu.async_copy(table_h.at[src_v], rows_v, sem).wait()
            pltpu.sync_copy(rows_v, acc.at[dst_v], add=True)
            return carry

        lax.fori_loop(0, NCHUNK, body, 0)
        plsc.subcore_barrier()
        pltpu.sync_copy(acc.at[pl.ds(s * RPS, RPS)],
                        out_h.at[pl.ds(c * N + s * RPS, RPS)])

        @pl.when(s == NSUB - 1)
        def _():
            pltpu.sync_copy(acc.at[pl.ds(TOFF, TAIL)],
                            out_h.at[pl.ds(c * N + TOFF, TAIL)])

    return k(table, src_b, dst_loc, zeros)


R1 = 400
R2 = 2000


def _fuse_body(x_ref, news_ref, deg_ref, w0_ref, w1_ref, w2_ref, cb_ref,
               npw1_ref, npb1_ref, npw2_ref, npb2_ref, xaug_ref, dinv_ref):
    xb = x_ref[...]
    z = jnp.zeros((R1, 1), jnp.float32)
    xm1 = jnp.concatenate([z, xb[:, :-1]], axis=1)
    xp1 = jnp.concatenate([xb[:, 1:], z], axis=1)
    h = (xm1[:, :, None] * w0_ref[...][0][None, None, :]
         + xb[:, :, None] * w1_ref[...][0][None, None, :]
         + xp1[:, :, None] * w2_ref[...][0][None, None, :]
         + cb_ref[...][0][None, None, :])
    cnn = jnp.mean(jax.nn.relu(h), axis=1)
    nh = jax.nn.relu(jnp.dot(news_ref[...], npw1_ref[...],
                             preferred_element_type=jnp.float32, precision=lax.Precision.HIGHEST) + npb1_ref[...])
    nf = jnp.dot(nh, npw2_ref[...], preferred_element_type=jnp.float32, precision=lax.Precision.HIGHEST) + npb2_ref[...]
    dinv = lax.rsqrt(deg_ref[...])
    col = lax.broadcasted_iota(jnp.int32, (R1, 16), 1)
    xaug_ref[:, 0:16] = jnp.where(col == 0, dinv, 0.0)
    xaug_ref[:, 16:80] = cnn * dinv
    xaug_ref[:, 80:112] = nf * dinv
    xaug_ref[:, 112:128] = jnp.zeros((R1, 16), jnp.float32)
    dinv_ref[...] = dinv


def _layer_body(agg_ref, xaug_ref, dinv_ref, what_ref, gw2_ref, gb2_ref, out_ref):
    dinv = dinv_ref[...]
    zz = (agg_ref[...] + xaug_ref[...]) * dinv
    h1 = jax.nn.relu(jnp.dot(zz, what_ref[...], preferred_element_type=jnp.float32, precision=lax.Precision.HIGHEST))
    m = jnp.dot(h1, gw2_ref[...], preferred_element_type=jnp.float32, precision=lax.Precision.HIGHEST) + gb2_ref[...]
    out_ref[...] = m * dinv


def _head_body(agg2_ref, mp_ref, dinv_ref, mw1_ref, mb1_ref, mw2_ref, mb2_ref, out_ref):
    o2 = (agg2_ref[...] + mp_ref[...]) * dinv_ref[...]
    h = jax.nn.relu(jnp.dot(o2, mw1_ref[...], preferred_element_type=jnp.float32, precision=lax.Precision.HIGHEST)
                    + mb1_ref[...])
    out_ref[...] = jnp.dot(h, mw2_ref[...], preferred_element_type=jnp.float32, precision=lax.Precision.HIGHEST) + mb2_ref[...]


def _full(shape):
    return pl.BlockSpec(shape, lambda i: (0, 0))


def kernel(price_data_x, edge_index, news_features, conv_w, conv_b,
           npw1, npb1, npw2, npb2, gw1, gb1, gw2, gb2, mw1, mb1, mw2, mb2):
    x = price_data_x.reshape(B * N, L)
    news = news_features.reshape(B * N, NEWS_DIM)
    src = edge_index[0]
    dst = edge_index[1]
    src_b = jnp.concatenate([src, src + N])

    ones_t = jnp.ones((B * N, 128), jnp.float32)
    deg_out = _seg_sum(ones_t, src_b, dst, 128)
    deg = deg_out[:, :1] + 1.0

    grid1 = (B * N) // R1
    w0 = conv_w[:, 0, 0].reshape(1, CNN_C)
    w1 = conv_w[:, 0, 1].reshape(1, CNN_C)
    w2 = conv_w[:, 0, 2].reshape(1, CNN_C)
    cb = conv_b.reshape(1, CNN_C)
    xaug, dinv = pl.pallas_call(
        _fuse_body,
        grid=(grid1,),
        in_specs=[
            pl.BlockSpec((R1, L), lambda i: (i, 0)),
            pl.BlockSpec((R1, NEWS_DIM), lambda i: (i, 0)),
            pl.BlockSpec((R1, 1), lambda i: (i, 0)),
            _full((1, CNN_C)), _full((1, CNN_C)), _full((1, CNN_C)), _full((1, CNN_C)),
            _full((NEWS_DIM, 2 * NEWS_P)), _full((1, 2 * NEWS_P)),
            _full((2 * NEWS_P, NEWS_P)), _full((1, NEWS_P)),
        ],
        out_specs=[
            pl.BlockSpec((R1, D1), lambda i: (i, 0)),
            pl.BlockSpec((R1, 1), lambda i: (i, 0)),
        ],
        out_shape=[
            jax.ShapeDtypeStruct((B * N, D1), jnp.float32),
            jax.ShapeDtypeStruct((B * N, 1), jnp.float32),
        ],
    )(x, news, deg, w0, w1, w2, cb,
      npw1, npb1.reshape(1, -1), npw2, npb2.reshape(1, -1))

    agg1 = _seg_sum(xaug, src_b, dst, D1)

    w_hat = jnp.zeros((D1, G_HID), jnp.float32).at[0, :].set(gb1).at[16:112, :].set(gw1)
    grid2 = (B * N) // R2
    mp = pl.pallas_call(
        _layer_body,
        grid=(grid2,),
        in_specs=[
            pl.BlockSpec((R2, D1), lambda i: (i, 0)),
            pl.BlockSpec((R2, D1), lambda i: (i, 0)),
            pl.BlockSpec((R2, 1), lambda i: (i, 0)),
            _full((D1, G_HID)), _full((G_HID, D2)), _full((1, D2)),
        ],
        out_specs=pl.BlockSpec((R2, D2), lambda i: (i, 0)),
        out_shape=jax.ShapeDtypeStruct((B * N, D2), jnp.float32),
    )(agg1, xaug, dinv, w_hat, gw2, gb2.reshape(1, -1))

    agg2 = _seg_sum(mp, src_b, dst, D2)

    mw2_pad = jnp.zeros((D2, 128), jnp.float32).at[:, :2].set(mw2)
    mb2_pad = jnp.zeros((1, 128), jnp.float32).at[0, :2].set(mb2)
    out = pl.pallas_call(
        _head_body,
        grid=(grid2,),
        in_specs=[
            pl.BlockSpec((R2, D2), lambda i: (i, 0)),
            pl.BlockSpec((R2, D2), lambda i: (i, 0)),
            pl.BlockSpec((R2, 1), lambda i: (i, 0)),
            _full((D2, 128)), _full((1, 128)), _full((D2, 128)), _full((1, 128)),
        ],
        out_specs=pl.BlockSpec((R2, 128), lambda i: (i, 0)),
        out_shape=jax.ShapeDtypeStruct((B * N, 128), jnp.float32),
    )(agg2, mp, dinv, mw1, mb1.reshape(1, -1), mw2_pad, mb2_pad)

    return out[:, :2].reshape(B, N, 2)


# speedup vs baseline: 8.7509x; 1.6539x over previous
import functools

import jax
import jax.numpy as jnp
from jax import lax
from jax.experimental import pallas as pl
from jax.experimental.pallas import tpu as pltpu
from jax.experimental.pallas import tpu_sc as plsc

B = 2
N = 10000
E = 160000
L = 64
NEWS_DIM = 128
CNN_C = 64
NEWS_P = 32
D1 = 128
D2 = 128
G_HID = 256
NSUB = 16
EPS = E // NSUB
CH = 80
NCHUNK = EPS // CH
RPS = 624
TAIL = N - NSUB * RPS
TOFF = NSUB * RPS


def _seg_sum(table, src_b, dst_loc, d):
    zeros = jnp.zeros((N, d), jnp.float32)
    mesh = plsc.VectorSubcoreMesh(core_axis_name="c", subcore_axis_name="s")

    @functools.partial(
        pl.kernel,
        mesh=mesh,
        out_type=jax.ShapeDtypeStruct((B * N, d), jnp.float32),
        scratch_types=[
            pltpu.VMEM((EPS,), jnp.int32),
            pltpu.VMEM((EPS,), jnp.int32),
            pltpu.VMEM((CH,), jnp.int32),
            pltpu.VMEM((CH, d), jnp.float32),
            pltpu.VMEM((CH, d), jnp.float32),
            pltpu.VMEM_SHARED((N, d), jnp.float32),
            pltpu.SemaphoreType.DMA,
        ],
    )
    def k(table_h, srcb_h, dstl_h, zeros_h, out_h,
          src_all, dst_all, dst_v, rows0, rows1, acc, sem):
        c = lax.axis_index("c")
        s = lax.axis_index("s")
        pltpu.sync_copy(zeros_h.at[pl.ds(s * RPS, RPS)], acc.at[pl.ds(s * RPS, RPS)])
        pltpu.sync_copy(srcb_h.at[pl.ds(c * E + s * EPS, EPS)], src_all)
        pltpu.sync_copy(dstl_h.at[pl.ds(s * EPS, EPS)], dst_all)

        @pl.when(s == NSUB - 1)
        def _():
            pltpu.sync_copy(zeros_h.at[pl.ds(TOFF, TAIL)], acc.at[pl.ds(TOFF, TAIL)])

        plsc.subcore_barrier()

        def issue(ci, buf):
            pltpu.async_copy(table_h.at[src_all.at[pl.ds(ci * CH, CH)]], buf, sem)

        def wait(buf):
            pltpu.make_async_copy(table_h.at[src_all.at[pl.ds(0, CH)]], buf, sem).wait()

        def scatter(ci, buf):
            for j in range(CH // 16):
                dst_v[pl.ds(j * 16, 16)] = dst_all[pl.ds(ci * CH + j * 16, 16)]
            pltpu.sync_copy(buf, acc.at[dst_v], add=True)

        issue(0, rows0)

        def body2(p, carry):
            issue(2 * p + 1, rows1)
            wait(rows0)
            scatter(2 * p, rows0)
            issue(2 * p + 2, rows0)
            wait(rows1)
            scatter(2 * p + 1, rows1)
            return carry

        lax.fori_loop(0, (NCHUNK - 1) // 2, body2, 0)
        wait(rows0)
        scatter(NCHUNK - 1, rows0)
        plsc.subcore_barrier()
        pltpu.sync_copy(acc.at[pl.ds(s * RPS, RPS)],
                        out_h.at[pl.ds(c * N + s * RPS, RPS)])

        @pl.when(s == NSUB - 1)
        def _():
            pltpu.sync_copy(acc.at[pl.ds(TOFF, TAIL)],
                            out_h.at[pl.ds(c * N + TOFF, TAIL)])

    return k(table, src_b, dst_loc, zeros)


DDEG = 16


def _deg_count(dst_loc):
    zeros = jnp.zeros((N, DDEG), jnp.float32)
    ones_blk = jnp.ones((CH, DDEG), jnp.float32)
    mesh = plsc.VectorSubcoreMesh(core_axis_name="c", subcore_axis_name="s")

    @functools.partial(
        pl.kernel,
        mesh=mesh,
        out_type=jax.ShapeDtypeStruct((B * N, DDEG), jnp.float32),
        scratch_types=[
            pltpu.VMEM((EPS,), jnp.int32),
            pltpu.VMEM((CH,), jnp.int32),
            pltpu.VMEM((CH, DDEG), jnp.float32),
            pltpu.VMEM_SHARED((N, DDEG), jnp.float32),
        ],
    )
    def k(dstl_h, zeros_h, ones_h, out_h, dst_all, dst_v, ones_v, acc):
        c = lax.axis_index("c")
        s = lax.axis_index("s")
        pltpu.sync_copy(zeros_h.at[pl.ds(s * RPS, RPS)], acc.at[pl.ds(s * RPS, RPS)])
        pltpu.sync_copy(dstl_h.at[pl.ds(s * EPS, EPS)], dst_all)
        pltpu.sync_copy(ones_h, ones_v)

        @pl.when(s == NSUB - 1)
        def _():
            pltpu.sync_copy(zeros_h.at[pl.ds(TOFF, TAIL)], acc.at[pl.ds(TOFF, TAIL)])

        plsc.subcore_barrier()

        def body(i, carry):
            for j in range(CH // 16):
                dst_v[pl.ds(j * 16, 16)] = dst_all[pl.ds(i * CH + j * 16, 16)]
            pltpu.sync_copy(ones_v, acc.at[dst_v], add=True)
            return carry

        lax.fori_loop(0, NCHUNK, body, 0)
        plsc.subcore_barrier()
        pltpu.sync_copy(acc.at[pl.ds(s * RPS, RPS)],
                        out_h.at[pl.ds(c * N + s * RPS, RPS)])

        @pl.when(s == NSUB - 1)
        def _():
            pltpu.sync_copy(acc.at[pl.ds(TOFF, TAIL)],
                            out_h.at[pl.ds(c * N + TOFF, TAIL)])

    return k(dst_loc, zeros, ones_blk)


R1 = 400
R2 = 2000


def _fuse_body(x_ref, news_ref, deg_ref, w0_ref, w1_ref, w2_ref, cb_ref,
               npw1_ref, npb1_ref, npw2_ref, npb2_ref, xaug_ref, dinv_ref):
    xb = x_ref[...]
    z = jnp.zeros((R1, 1), jnp.float32)
    xm1 = jnp.concatenate([z, xb[:, :-1]], axis=1)
    xp1 = jnp.concatenate([xb[:, 1:], z], axis=1)
    h = (xm1[:, :, None] * w0_ref[...][0][None, None, :]
         + xb[:, :, None] * w1_ref[...][0][None, None, :]
         + xp1[:, :, None] * w2_ref[...][0][None, None, :]
         + cb_ref[...][0][None, None, :])
    cnn = jnp.mean(jax.nn.relu(h), axis=1)
    nh = jax.nn.relu(jnp.dot(news_ref[...], npw1_ref[...],
                             preferred_element_type=jnp.float32, precision=lax.Precision.HIGHEST) + npb1_ref[...])
    nf = jnp.dot(nh, npw2_ref[...], preferred_element_type=jnp.float32, precision=lax.Precision.HIGHEST) + npb2_ref[...]
    dinv = lax.rsqrt(deg_ref[...])
    col = lax.broadcasted_iota(jnp.int32, (R1, 16), 1)
    xaug_ref[:, 0:16] = jnp.where(col == 0, dinv, 0.0)
    xaug_ref[:, 16:80] = cnn * dinv
    xaug_ref[:, 80:112] = nf * dinv
    xaug_ref[:, 112:128] = jnp.zeros((R1, 16), jnp.float32)
    dinv_ref[...] = dinv


def _layer_body(agg_ref, xaug_ref, dinv_ref, what_ref, gw2_ref, gb2_ref, out_ref):
    dinv = dinv_ref[...]
    zz = (agg_ref[...] + xaug_ref[...]) * dinv
    h1 = jax.nn.relu(jnp.dot(zz, what_ref[...], preferred_element_type=jnp.float32, precision=lax.Precision.HIGHEST))
    m = jnp.dot(h1, gw2_ref[...], preferred_element_type=jnp.float32, precision=lax.Precision.HIGHEST) + gb2_ref[...]
    out_ref[...] = m * dinv


def _head_body(agg2_ref, mp_ref, dinv_ref, mw1_ref, mb1_ref, mw2_ref, mb2_ref, out_ref):
    o2 = (agg2_ref[...] + mp_ref[...]) * dinv_ref[...]
    h = jax.nn.relu(jnp.dot(o2, mw1_ref[...], preferred_element_type=jnp.float32, precision=lax.Precision.HIGHEST)
                    + mb1_ref[...])
    out_ref[...] = jnp.dot(h, mw2_ref[...], preferred_element_type=jnp.float32, precision=lax.Precision.HIGHEST) + mb2_ref[...]


def _full(shape):
    return pl.BlockSpec(shape, lambda i: (0, 0))


def kernel(price_data_x, edge_index, news_features, conv_w, conv_b,
           npw1, npb1, npw2, npb2, gw1, gb1, gw2, gb2, mw1, mb1, mw2, mb2):
    x = price_data_x.reshape(B * N, L)
    news = news_features.reshape(B * N, NEWS_DIM)
    src = edge_index[0]
    dst = edge_index[1]
    src_b = jnp.concatenate([src, src + N])

    deg_out = _deg_count(dst)
    deg = deg_out[:, :1] + 1.0

    grid1 = (B * N) // R1
    w0 = conv_w[:, 0, 0].reshape(1, CNN_C)
    w1 = conv_w[:, 0, 1].reshape(1, CNN_C)
    w2 = conv_w[:, 0, 2].reshape(1, CNN_C)
    cb = conv_b.reshape(1, CNN_C)
    xaug, dinv = pl.pallas_call(
        _fuse_body,
        grid=(grid1,),
        in_specs=[
            pl.BlockSpec((R1, L), lambda i: (i, 0)),
            pl.BlockSpec((R1, NEWS_DIM), lambda i: (i, 0)),
            pl.BlockSpec((R1, 1), lambda i: (i, 0)),
            _full((1, CNN_C)), _full((1, CNN_C)), _full((1, CNN_C)), _full((1, CNN_C)),
            _full((NEWS_DIM, 2 * NEWS_P)), _full((1, 2 * NEWS_P)),
            _full((2 * NEWS_P, NEWS_P)), _full((1, NEWS_P)),
        ],
        out_specs=[
            pl.BlockSpec((R1, D1), lambda i: (i, 0)),
            pl.BlockSpec((R1, 1), lambda i: (i, 0)),
        ],
        out_shape=[
            jax.ShapeDtypeStruct((B * N, D1), jnp.float32),
            jax.ShapeDtypeStruct((B * N, 1), jnp.float32),
        ],
    )(x, news, deg, w0, w1, w2, cb,
      npw1, npb1.reshape(1, -1), npw2, npb2.reshape(1, -1))

    agg1 = _seg_sum(xaug, src_b, dst, D1)

    w_hat = jnp.zeros((D1, G_HID), jnp.float32).at[0, :].set(gb1).at[16:112, :].set(gw1)
    grid2 = (B * N) // R2
    mp = pl.pallas_call(
        _layer_body,
        grid=(grid2,),
        in_specs=[
            pl.BlockSpec((R2, D1), lambda i: (i, 0)),
            pl.BlockSpec((R2, D1), lambda i: (i, 0)),
            pl.BlockSpec((R2, 1), lambda i: (i, 0)),
            _full((D1, G_HID)), _full((G_HID, D2)), _full((1, D2)),
        ],
        out_specs=pl.BlockSpec((R2, D2), lambda i: (i, 0)),
        out_shape=jax.ShapeDtypeStruct((B * N, D2), jnp.float32),
    )(agg1, xaug, dinv, w_hat, gw2, gb2.reshape(1, -1))

    agg2 = _seg_sum(mp, src_b, dst, D2)

    mw2_pad = jnp.zeros((D2, 128), jnp.float32).at[:, :2].set(mw2)
    mb2_pad = jnp.zeros((1, 128), jnp.float32).at[0, :2].set(mb2)
    out = pl.pallas_call(
        _head_body,
        grid=(grid2,),
        in_specs=[
            pl.BlockSpec((R2, D2), lambda i: (i, 0)),
            pl.BlockSpec((R2, D2), lambda i: (i, 0)),
            pl.BlockSpec((R2, 1), lambda i: (i, 0)),
            _full((D2, 128)), _full((1, 128)), _full((D2, 128)), _full((1, 128)),
        ],
        out_specs=pl.BlockSpec((R2, 128), lambda i: (i, 0)),
        out_shape=jax.ShapeDtypeStruct((B * N, 128), jnp.float32),
    )(agg2, mp, dinv, mw1, mb1.reshape(1, -1), mw2_pad, mb2_pad)

    return out[:, :2].reshape(B, N, 2)
